# baseline (speedup 1.0000x reference)
import functools
import math

import jax
import jax.numpy as jnp
from jax.experimental import pallas as pl
from jax.experimental.pallas import tpu as pltpu

B, L, Dm, Hq, Hkv, Dh, E, K, F = 1, 2048, 2048, 32, 4, 128, 16, 8, 128
EPS = 1e-06
NH = Hq + 2 * Hkv
REP = Hq // Hkv

BL = 512
BQ = 512
BK = 512
BT = 512
VA = 2 * Dh


def _dot(a, b):
    return jax.lax.dot_general(a, b, (((1,), (0,)), ((), ())),
                               preferred_element_type=jnp.float32)


def _dot_nt(a, b):
    return jax.lax.dot_general(a, b, (((1,), (1,)), ((), ())),
                               preferred_element_type=jnp.float32)


def _qkv_kernel(x_ref, w_ref, na_ref, normw_ref, cos_ref, sins_ref, out_ref,
                h_scr, y_scr):
    j = pl.program_id(1)

    @pl.when(j == 0)
    def _():
        x = x_ref[...]
        ms = jnp.mean(x * x, axis=-1, keepdims=True)
        h_scr[...] = (x * jax.lax.rsqrt(ms + EPS) * na_ref[...]
                      ).astype(jnp.bfloat16)

    @pl.when((0 < j) & (j <= Hq + Hkv))
    def _():
        y = y_scr[...]
        jm = jnp.full((Dh, Dh), 1.0 / Dh, jnp.bfloat16)
        msy = _dot((y * y).astype(jnp.bfloat16), jm)
        yn = y * jax.lax.rsqrt(msy + EPS) * normw_ref[...]
        rot = jnp.roll(yn, -(Dh // 2), axis=1)
        out_ref[0] = (cos_ref[...] * yn + sins_ref[...] * rot
                      ).astype(jnp.bfloat16)

    @pl.when(j > Hq + Hkv)
    def _():
        out_ref[0] = y_scr[...].astype(jnp.bfloat16)

    @pl.when(j < NH)
    def _():
        y_scr[...] = _dot(h_scr[...], w_ref[...])


def _attn_kernel(q_ref, k_ref, vaug_ref, mask_ref, o_ref, acc_scr):
    i = pl.program_id(1)
    q = q_ref[0]
    acc_scr[...] = jnp.zeros((BQ, VA), jnp.float32)

    def body(j, _):
        kb = k_ref[0, pl.ds(j * BK, BK), :]
        p = jnp.exp(_dot_nt(q, kb)).astype(jnp.bfloat16)
        vb = vaug_ref[0, pl.ds(j * BK, BK), :]
        acc_scr[...] += _dot(p, vb)
        return 0

    jax.lax.fori_loop(0, i, body, 0)
    kb = k_ref[0, pl.ds(i * BK, BK), :]
    p = (jnp.exp(_dot_nt(q, kb)) * mask_ref[...]).astype(jnp.bfloat16)
    acc = acc_scr[...] + _dot(p, vaug_ref[0, pl.ds(i * BK, BK), :])
    o_ref[...] = (acc[:, :Dh] / acc[:, Dh:Dh + 1]).astype(jnp.bfloat16)


def _oproj_gate_kernel(o_ref, wo_ref, x_ref, nm_ref, wg_ref,
                       xa_ref, h2_ref, wfull_ref):
    xa = x_ref[...] + _dot(o_ref[...], wo_ref[...])
    xa_ref[...] = xa
    ms = jnp.mean(xa * xa, axis=-1, keepdims=True)
    h2 = xa * jax.lax.rsqrt(ms + EPS) * nm_ref[...]
    h2_ref[...] = h2.astype(jnp.bfloat16)
    logits = _dot(h2, wg_ref[...])
    mx = jnp.max(logits, axis=-1, keepdims=True)
    ex = jnp.exp(logits - mx)
    probs = ex / jnp.sum(ex, axis=-1, keepdims=True)
    a = probs[:, None, :]
    b = probs[:, :, None]
    ij = jax.lax.broadcasted_iota(jnp.int32, (BT, E, E), 2)
    ie = jax.lax.broadcasted_iota(jnp.int32, (BT, E, E), 1)
    cmp = (a > b) | ((a == b) & (ij < ie))
    rank = jnp.sum(cmp.astype(jnp.float32), axis=2)
    w = jnp.where(rank < K, probs, 0.0)
    wfull_ref[...] = w / jnp.sum(w, axis=-1, keepdims=True)


def _moe_kernel(h2_ref, wg_ref, wu_ref, wd_ref, wfull_ref, sel_ref, xa_ref,
                out_ref):
    h2 = h2_ref[...]
    g = _dot(h2, wg_ref[...])
    u = _dot(h2, wu_ref[...])
    act = g * jax.lax.logistic(g) * u
    wexp = _dot(wfull_ref[...], sel_ref[...])
    down_in = (act * wexp).astype(jnp.bfloat16)
    out_ref[...] = xa_ref[...] + _dot(down_in, wd_ref[...])


@functools.partial(jax.jit, static_argnames=("interpret",))
def kernel(x, cos, sin, Wq, Wk, Wv, Wo, qn, kn, na, nm, Wgate, Wg, Wu, Wd,
           interpret=False):
    f32 = jnp.float32
    bf16 = jnp.bfloat16
    x2 = x[0]
    cos2 = cos[0]
    sign = jnp.concatenate([-jnp.ones((Dh // 2,), f32),
                            jnp.ones((Dh // 2,), f32)])
    sins2 = sin[0] * sign
    wqkv = jnp.concatenate([Wq, Wk, Wv], axis=1).astype(bf16)
    normw = jnp.concatenate(
        [jnp.tile(qn * (1.0 / math.sqrt(Dh)), Hq), jnp.tile(kn, Hkv),
         jnp.ones((Hkv * Dh,), f32)]).reshape(1, NH * Dh)
    na2 = na.reshape(1, Dm)
    nm2 = nm.reshape(1, Dm)
    wo_b = Wo.astype(bf16)
    wg_all = jnp.transpose(Wg, (1, 0, 2)).reshape(Dm, E * F).astype(bf16)
    wu_all = jnp.transpose(Wu, (1, 0, 2)).reshape(Dm, E * F).astype(bf16)
    wd_all = Wd.reshape(E * F, Dm).astype(bf16)
    sel = jnp.repeat(jnp.eye(E, dtype=f32), F, axis=1).reshape(E, E * F)
    tri = jnp.tril(jnp.ones((BQ, BK), f32))

    qkv = pl.pallas_call(
        _qkv_kernel,
        grid=(L // BL, NH + 1),
        in_specs=[
            pl.BlockSpec((BL, Dm), lambda i, j: (i, 0)),
            pl.BlockSpec((Dm, Dh), lambda i, j: (0, jnp.minimum(j, NH - 1))),
            pl.BlockSpec((1, Dm), lambda i, j: (0, 0)),
            pl.BlockSpec((1, Dh), lambda i, j: (0, jnp.maximum(j - 1, 0))),
            pl.BlockSpec((BL, Dh), lambda i, j: (i, 0)),
            pl.BlockSpec((BL, Dh), lambda i, j: (i, 0)),
        ],
        out_specs=pl.BlockSpec((1, BL, Dh),
                               lambda i, j: (jnp.maximum(j - 1, 0), i, 0)),
        out_shape=jax.ShapeDtypeStruct((NH, L, Dh), bf16),
        scratch_shapes=[pltpu.VMEM((BL, Dm), bf16),
                        pltpu.VMEM((BL, Dh), f32)],
        compiler_params=pltpu.CompilerParams(
            dimension_semantics=("parallel", "arbitrary")),
        interpret=interpret,
    )(x2, wqkv, na2, normw, cos2, sins2)

    vv = qkv[Hq + Hkv:]
    vaug = jnp.concatenate(
        [vv, jnp.ones((Hkv, L, 1), bf16),
         jnp.zeros((Hkv, L, VA - Dh - 1), bf16)], axis=2)

    o = pl.pallas_call(
        _attn_kernel,
        grid=(Hq, L // BQ),
        in_specs=[
            pl.BlockSpec((1, BQ, Dh), lambda h, i: (h, i, 0)),
            pl.BlockSpec((1, L, Dh), lambda h, i: (Hq + h // REP, 0, 0)),
            pl.BlockSpec((1, L, VA), lambda h, i: (h // REP, 0, 0)),
            pl.BlockSpec((BQ, BK), lambda h, i: (0, 0)),
        ],
        out_specs=pl.BlockSpec((BQ, Dh), lambda h, i: (i, h)),
        out_shape=jax.ShapeDtypeStruct((L, Hq * Dh), bf16),
        scratch_shapes=[pltpu.VMEM((BQ, VA), f32)],
        compiler_params=pltpu.CompilerParams(
            dimension_semantics=("parallel", "parallel")),
        interpret=interpret,
    )(qkv, qkv, vaug, tri)

    xa, h2, wfull = pl.pallas_call(
        _oproj_gate_kernel,
        grid=(L // BT,),
        in_specs=[
            pl.BlockSpec((BT, Hq * Dh), lambda i: (i, 0)),
            pl.BlockSpec((Hq * Dh, Dm), lambda i: (0, 0)),
            pl.BlockSpec((BT, Dm), lambda i: (i, 0)),
            pl.BlockSpec((1, Dm), lambda i: (0, 0)),
            pl.BlockSpec((Dm, E), lambda i: (0, 0)),
        ],
        out_specs=[
            pl.BlockSpec((BT, Dm), lambda i: (i, 0)),
            pl.BlockSpec((BT, Dm), lambda i: (i, 0)),
            pl.BlockSpec((BT, E), lambda i: (i, 0)),
        ],
        out_shape=[
            jax.ShapeDtypeStruct((L, Dm), f32),
            jax.ShapeDtypeStruct((L, Dm), bf16),
            jax.ShapeDtypeStruct((L, E), f32),
        ],
        compiler_params=pltpu.CompilerParams(
            dimension_semantics=("parallel",)),
        interpret=interpret,
    )(o, wo_b, x2, nm2, Wgate)

    out = pl.pallas_call(
        _moe_kernel,
        grid=(L // BT,),
        in_specs=[
            pl.BlockSpec((BT, Dm), lambda i: (i, 0)),
            pl.BlockSpec((Dm, E * F), lambda i: (0, 0)),
            pl.BlockSpec((Dm, E * F), lambda i: (0, 0)),
            pl.BlockSpec((E * F, Dm), lambda i: (0, 0)),
            pl.BlockSpec((BT, E), lambda i: (i, 0)),
            pl.BlockSpec((E, E * F), lambda i: (0, 0)),
            pl.BlockSpec((BT, Dm), lambda i: (i, 0)),
        ],
        out_specs=pl.BlockSpec((BT, Dm), lambda i: (i, 0)),
        out_shape=jax.ShapeDtypeStruct((L, Dm), f32),
        compiler_params=pltpu.CompilerParams(
            dimension_semantics=("parallel",)),
        interpret=interpret,
    )(h2, wg_all, wu_all, wd_all, wfull, sel, xa)

    return out.reshape(B, L, Dm)


# speedup vs baseline: 1.0000x; 1.0000x over previous
import functools
import math

import jax
import jax.numpy as jnp
from jax.experimental import pallas as pl
from jax.experimental.pallas import tpu as pltpu

B, L, Dm, Hq, Hkv, Dh, E, K, F = 1, 2048, 2048, 32, 4, 128, 16, 8, 128
EPS = 1e-06
NH = Hq + 2 * Hkv
REP = Hq // Hkv

BL = 512
BQ = 512
BK = 512
BT = 512
VA = 2 * Dh


def _dot(a, b):
    return jax.lax.dot_general(a, b, (((1,), (0,)), ((), ())),
                               preferred_element_type=jnp.float32)


def _dot_nt(a, b):
    return jax.lax.dot_general(a, b, (((1,), (1,)), ((), ())),
                               preferred_element_type=jnp.float32)


def _qkv_kernel(x_ref, w_ref, na_ref, normw_ref, cos_ref, sins_ref, out_ref,
                h_scr, y_scr):
    j = pl.program_id(1)

    @pl.when(j == 0)
    def _():
        x = x_ref[...]
        ms = jnp.mean(x * x, axis=-1, keepdims=True)
        h_scr[...] = (x * jax.lax.rsqrt(ms + EPS) * na_ref[...]
                      ).astype(jnp.bfloat16)

    @pl.when((0 < j) & (j <= Hq + Hkv))
    def _():
        y = y_scr[...]
        jm = jnp.full((Dh, Dh), 1.0 / Dh, jnp.bfloat16)
        msy = _dot((y * y).astype(jnp.bfloat16), jm)
        yn = y * jax.lax.rsqrt(msy + EPS) * normw_ref[...]
        rot = jnp.roll(yn, -(Dh // 2), axis=1)
        out_ref[0] = (cos_ref[...] * yn + sins_ref[...] * rot
                      ).astype(jnp.bfloat16)

    @pl.when(j > Hq + Hkv)
    def _():
        out_ref[0] = y_scr[...].astype(jnp.bfloat16)

    @pl.when(j < NH)
    def _():
        y_scr[...] = _dot(h_scr[...], w_ref[...])


def _attn_kernel(q_ref, k_ref, vaug_ref, mask_ref, o_ref, acc_scr):
    i = pl.program_id(1)
    q = q_ref[0]
    acc_scr[...] = jnp.zeros((BQ, VA), jnp.float32)

    def body(j, _):
        kb = k_ref[0, pl.ds(j * BK, BK), :]
        p = jnp.exp(_dot_nt(q, kb)).astype(jnp.bfloat16)
        vb = vaug_ref[0, pl.ds(j * BK, BK), :]
        acc_scr[...] += _dot(p, vb)
        return 0

    jax.lax.fori_loop(0, i, body, 0)
    kb = k_ref[0, pl.ds(i * BK, BK), :]
    p = (jnp.exp(_dot_nt(q, kb)) * mask_ref[...]).astype(jnp.bfloat16)
    acc = acc_scr[...] + _dot(p, vaug_ref[0, pl.ds(i * BK, BK), :])
    o_ref[...] = (acc[:, :Dh] / acc[:, Dh:Dh + 1]).astype(jnp.bfloat16)


def _oproj_gate_kernel(o_ref, wo_ref, x_ref, nm_ref, wg_ref,
                       xa_ref, h2_ref, wfull_ref):
    xa = x_ref[...] + _dot(o_ref[...], wo_ref[...])
    xa_ref[...] = xa
    ms = jnp.mean(xa * xa, axis=-1, keepdims=True)
    h2 = xa * jax.lax.rsqrt(ms + EPS) * nm_ref[...]
    h2_ref[...] = h2.astype(jnp.bfloat16)
    logits = _dot(h2, wg_ref[...])
    mx = jnp.max(logits, axis=-1, keepdims=True)
    ex = jnp.exp(logits - mx)
    probs = ex / jnp.sum(ex, axis=-1, keepdims=True)
    a = probs[:, None, :]
    b = probs[:, :, None]
    ij = jax.lax.broadcasted_iota(jnp.int32, (BT, E, E), 2)
    ie = jax.lax.broadcasted_iota(jnp.int32, (BT, E, E), 1)
    cmp = (a > b) | ((a == b) & (ij < ie))
    rank = jnp.sum(cmp.astype(jnp.float32), axis=2)
    w = jnp.where(rank < K, probs, 0.0)
    wfull_ref[...] = w / jnp.sum(w, axis=-1, keepdims=True)


def _moe_kernel(h2_ref, wg_ref, wu_ref, wd_ref, wfull_ref, sel_ref, xa_ref,
                out_ref):
    h2 = h2_ref[...]
    g = _dot(h2, wg_ref[...])
    u = _dot(h2, wu_ref[...])
    act = g * jax.lax.logistic(g) * u
    wexp = _dot(wfull_ref[...], sel_ref[...])
    down_in = (act * wexp).astype(jnp.bfloat16)
    out_ref[...] = xa_ref[...] + _dot(down_in, wd_ref[...])


def _qkv_call(x2, wqkv, na2, normw, cos2, sins2, interpret=False):
    f32 = jnp.float32
    bf16 = jnp.bfloat16
    return pl.pallas_call(
        _qkv_kernel,
        grid=(L // BL, NH + 1),
        in_specs=[
            pl.BlockSpec((BL, Dm), lambda i, j: (i, 0)),
            pl.BlockSpec((Dm, Dh), lambda i, j: (0, jnp.minimum(j, NH - 1))),
            pl.BlockSpec((1, Dm), lambda i, j: (0, 0)),
            pl.BlockSpec((1, Dh), lambda i, j: (0, jnp.maximum(j - 1, 0))),
            pl.BlockSpec((BL, Dh), lambda i, j: (i, 0)),
            pl.BlockSpec((BL, Dh), lambda i, j: (i, 0)),
        ],
        out_specs=pl.BlockSpec((1, BL, Dh),
                               lambda i, j: (jnp.maximum(j - 1, 0), i, 0)),
        out_shape=jax.ShapeDtypeStruct((NH, L, Dh), bf16),
        scratch_shapes=[pltpu.VMEM((BL, Dm), bf16),
                        pltpu.VMEM((BL, Dh), f32)],
        compiler_params=pltpu.CompilerParams(
            dimension_semantics=("parallel", "arbitrary")),
        interpret=interpret,
    )(x2, wqkv, na2, normw, cos2, sins2)


def _attn_call(qkv, vaug, tri, interpret=False):
    f32 = jnp.float32
    bf16 = jnp.bfloat16
    return pl.pallas_call(
        _attn_kernel,
        grid=(Hq, L // BQ),
        in_specs=[
            pl.BlockSpec((1, BQ, Dh), lambda h, i: (h, i, 0)),
            pl.BlockSpec((1, L, Dh), lambda h, i: (Hq + h // REP, 0, 0)),
            pl.BlockSpec((1, L, VA), lambda h, i: (h // REP, 0, 0)),
            pl.BlockSpec((BQ, BK), lambda h, i: (0, 0)),
        ],
        out_specs=pl.BlockSpec((BQ, Dh), lambda h, i: (i, h)),
        out_shape=jax.ShapeDtypeStruct((L, Hq * Dh), bf16),
        scratch_shapes=[pltpu.VMEM((BQ, VA), f32)],
        compiler_params=pltpu.CompilerParams(
            dimension_semantics=("parallel", "parallel")),
        interpret=interpret,
    )(qkv, qkv, vaug, tri)


def _oproj_call(o, wo_b, x2, nm2, Wgate, interpret=False):
    f32 = jnp.float32
    bf16 = jnp.bfloat16
    return pl.pallas_call(
        _oproj_gate_kernel,
        grid=(L // BT,),
        in_specs=[
            pl.BlockSpec((BT, Hq * Dh), lambda i: (i, 0)),
            pl.BlockSpec((Hq * Dh, Dm), lambda i: (0, 0)),
            pl.BlockSpec((BT, Dm), lambda i: (i, 0)),
            pl.BlockSpec((1, Dm), lambda i: (0, 0)),
            pl.BlockSpec((Dm, E), lambda i: (0, 0)),
        ],
        out_specs=[
            pl.BlockSpec((BT, Dm), lambda i: (i, 0)),
            pl.BlockSpec((BT, Dm), lambda i: (i, 0)),
            pl.BlockSpec((BT, E), lambda i: (i, 0)),
        ],
        out_shape=[
            jax.ShapeDtypeStruct((L, Dm), f32),
            jax.ShapeDtypeStruct((L, Dm), bf16),
            jax.ShapeDtypeStruct((L, E), f32),
        ],
        compiler_params=pltpu.CompilerParams(
            dimension_semantics=("parallel",)),
        interpret=interpret,
    )(o, wo_b, x2, nm2, Wgate)


def _moe_call(h2, wg_all, wu_all, wd_all, wfull, sel, xa, interpret=False):
    f32 = jnp.float32
    return pl.pallas_call(
        _moe_kernel,
        grid=(L // BT,),
        in_specs=[
            pl.BlockSpec((BT, Dm), lambda i: (i, 0)),
            pl.BlockSpec((Dm, E * F), lambda i: (0, 0)),
            pl.BlockSpec((Dm, E * F), lambda i: (0, 0)),
            pl.BlockSpec((E * F, Dm), lambda i: (0, 0)),
            pl.BlockSpec((BT, E), lambda i: (i, 0)),
            pl.BlockSpec((E, E * F), lambda i: (0, 0)),
            pl.BlockSpec((BT, Dm), lambda i: (i, 0)),
        ],
        out_specs=pl.BlockSpec((BT, Dm), lambda i: (i, 0)),
        out_shape=jax.ShapeDtypeStruct((L, Dm), f32),
        compiler_params=pltpu.CompilerParams(
            dimension_semantics=("parallel",)),
        interpret=interpret,
    )(h2, wg_all, wu_all, wd_all, wfull, sel, xa)


@functools.partial(jax.jit, static_argnames=("interpret",))
def kernel(x, cos, sin, Wq, Wk, Wv, Wo, qn, kn, na, nm, Wgate, Wg, Wu, Wd,
           interpret=False):
    f32 = jnp.float32
    bf16 = jnp.bfloat16
    x2 = x[0]
    cos2 = cos[0]
    sign = jnp.concatenate([-jnp.ones((Dh // 2,), f32),
                            jnp.ones((Dh // 2,), f32)])
    sins2 = sin[0] * sign
    wqkv = jnp.concatenate([Wq, Wk, Wv], axis=1).astype(bf16)
    normw = jnp.concatenate(
        [jnp.tile(qn * (1.0 / math.sqrt(Dh)), Hq), jnp.tile(kn, Hkv),
         jnp.ones((Hkv * Dh,), f32)]).reshape(1, NH * Dh)
    na2 = na.reshape(1, Dm)
    nm2 = nm.reshape(1, Dm)
    wo_b = Wo.astype(bf16)
    wg_all = jnp.transpose(Wg, (1, 0, 2)).reshape(Dm, E * F).astype(bf16)
    wu_all = jnp.transpose(Wu, (1, 0, 2)).reshape(Dm, E * F).astype(bf16)
    wd_all = Wd.reshape(E * F, Dm).astype(bf16)
    sel = jnp.repeat(jnp.eye(E, dtype=f32), F, axis=1).reshape(E, E * F)
    tri = jnp.tril(jnp.ones((BQ, BK), f32))

    qkv = _qkv_call(x2, wqkv, na2, normw, cos2, sins2, interpret=interpret)
    vv = qkv[Hq + Hkv:]
    vaug = jnp.concatenate(
        [vv, jnp.ones((Hkv, L, 1), bf16),
         jnp.zeros((Hkv, L, VA - Dh - 1), bf16)], axis=2)
    o = _attn_call(qkv, vaug, tri, interpret=interpret)
    xa, h2, wfull = _oproj_call(o, wo_b, x2, nm2, Wgate, interpret=interpret)
    out = _moe_call(h2, wg_all, wu_all, wd_all, wfull, sel, xa,
                    interpret=interpret)
    return out.reshape(B, L, Dm)


# speedup vs baseline: 1.0368x; 1.0367x over previous
import functools
import math

import jax
import jax.numpy as jnp
from jax.experimental import pallas as pl
from jax.experimental.pallas import tpu as pltpu

B, L, Dm, Hq, Hkv, Dh, E, K, F = 1, 2048, 2048, 32, 4, 128, 16, 8, 128
EPS = 1e-06
NH = Hq + 2 * Hkv
NQK = Hq + Hkv
REP = Hq // Hkv

BL = 1024
BQ = 512
BK = 512
BT = 256
BTM = 512
VA = 2 * Dh


def _dot(a, b):
    return jax.lax.dot_general(a, b, (((1,), (0,)), ((), ())),
                               preferred_element_type=jnp.float32)


def _dot_nt(a, b):
    return jax.lax.dot_general(a, b, (((1,), (1,)), ((), ())),
                               preferred_element_type=jnp.float32)


def _rms_kernel(x_ref, na_ref, h_ref):
    x = x_ref[...]
    ms = jnp.mean(x * x, axis=-1, keepdims=True)
    h_ref[...] = (x * jax.lax.rsqrt(ms + EPS) * na_ref[...]
                  ).astype(jnp.bfloat16)


def _qkvmm_kernel(h_ref, w_ref, y_ref):
    y_ref[0] = _dot(h_ref[...], w_ref[...]).astype(jnp.bfloat16)


def _rope_kernel(y_ref, normw_ref, cos_ref, sins_ref, out_ref):
    y = y_ref[0].astype(jnp.float32)
    jm = jnp.full((Dh, Dh), 1.0 / Dh, jnp.bfloat16)
    msy = _dot((y * y).astype(jnp.bfloat16), jm)
    yn = y * jax.lax.rsqrt(msy + EPS) * normw_ref[0]
    rot = jnp.roll(yn, -(Dh // 2), axis=1)
    out_ref[0] = (cos_ref[...] * yn + sins_ref[...] * rot
                  ).astype(jnp.bfloat16)


def _attn_kernel(q_ref, k_ref, vaug_ref, mask_ref, o_ref, acc_scr, s_scr):
    i = pl.program_id(1)
    q = q_ref[0]
    acc_scr[...] = jnp.zeros((BQ, VA), jnp.float32)
    s_scr[...] = _dot_nt(q, k_ref[0, pl.ds(0, BK), :])

    def body(j, _):
        p = jnp.exp(s_scr[...]).astype(jnp.bfloat16)
        vb = vaug_ref[0, pl.ds(j * BK, BK), :]
        kb1 = k_ref[0, pl.ds((j + 1) * BK, BK), :]
        acc_scr[...] += _dot(p, vb)
        s_scr[...] = _dot_nt(q, kb1)
        return 0

    jax.lax.fori_loop(0, i, body, 0)
    p = (jnp.exp(s_scr[...]) * mask_ref[...]).astype(jnp.bfloat16)
    acc = acc_scr[...] + _dot(p, vaug_ref[0, pl.ds(i * BK, BK), :])
    o_ref[...] = (acc[:, :Dh] / acc[:, Dh:Dh + 1]).astype(jnp.bfloat16)


def _oproj_gate_kernel(o_ref, wo_ref, x_ref, nm_ref, wg_ref,
                       xa_ref, h2_ref, wfull_ref):
    xa = x_ref[...] + _dot(o_ref[...], wo_ref[...])
    xa_ref[...] = xa
    ms = jnp.mean(xa * xa, axis=-1, keepdims=True)
    h2 = xa * jax.lax.rsqrt(ms + EPS) * nm_ref[...]
    h2_ref[...] = h2.astype(jnp.bfloat16)
    logits = _dot(h2, wg_ref[...])
    mx = jnp.max(logits, axis=-1, keepdims=True)
    ex = jnp.exp(logits - mx)
    probs = ex / jnp.sum(ex, axis=-1, keepdims=True)
    a = probs[:, None, :]
    b = probs[:, :, None]
    ij = jax.lax.broadcasted_iota(jnp.int32, (BT, E, E), 2)
    ie = jax.lax.broadcasted_iota(jnp.int32, (BT, E, E), 1)
    cmp = (a > b) | ((a == b) & (ij < ie))
    rank = jnp.sum(cmp.astype(jnp.float32), axis=2)
    w = jnp.where(rank < K, probs, 0.0)
    wfull_ref[...] = w / jnp.sum(w, axis=-1, keepdims=True)


def _moe_kernel(h2_ref, wg_ref, wu_ref, wd_ref, wfull_ref, sel_ref, xa_ref,
                out_ref):
    h2 = h2_ref[...]
    g = _dot(h2, wg_ref[...])
    u = _dot(h2, wu_ref[...])
    act = g * jax.lax.logistic(g) * u
    wexp = _dot(wfull_ref[...], sel_ref[...])
    down_in = (act * wexp).astype(jnp.bfloat16)
    out_ref[...] = xa_ref[...] + _dot(down_in, wd_ref[...])


def _rms_call(x2, na2, interpret=False):
    return pl.pallas_call(
        _rms_kernel,
        grid=(L // BL,),
        in_specs=[
            pl.BlockSpec((BL, Dm), lambda i: (i, 0)),
            pl.BlockSpec((1, Dm), lambda i: (0, 0)),
        ],
        out_specs=pl.BlockSpec((BL, Dm), lambda i: (i, 0)),
        out_shape=jax.ShapeDtypeStruct((L, Dm), jnp.bfloat16),
        compiler_params=pltpu.CompilerParams(
            dimension_semantics=("parallel",)),
        interpret=interpret,
    )(x2, na2)


def _qkvmm_call(hb, wqkv, interpret=False):
    return pl.pallas_call(
        _qkvmm_kernel,
        grid=(L // BL, NH),
        in_specs=[
            pl.BlockSpec((BL, Dm), lambda i, j: (i, 0)),
            pl.BlockSpec((Dm, Dh), lambda i, j: (0, j)),
        ],
        out_specs=pl.BlockSpec((1, BL, Dh), lambda i, j: (j, i, 0)),
        out_shape=jax.ShapeDtypeStruct((NH, L, Dh), jnp.bfloat16),
        compiler_params=pltpu.CompilerParams(
            dimension_semantics=("parallel", "parallel")),
        interpret=interpret,
    )(hb, wqkv)


def _rope_call(y, normw3, cos2, sins2, interpret=False):
    return pl.pallas_call(
        _rope_kernel,
        grid=(NQK, L // BL),
        in_specs=[
            pl.BlockSpec((1, BL, Dh), lambda j, i: (j, i, 0)),
            pl.BlockSpec((1, 1, Dh), lambda j, i: (j, 0, 0)),
            pl.BlockSpec((BL, Dh), lambda j, i: (i, 0)),
            pl.BlockSpec((BL, Dh), lambda j, i: (i, 0)),
        ],
        out_specs=pl.BlockSpec((1, BL, Dh), lambda j, i: (j, i, 0)),
        out_shape=jax.ShapeDtypeStruct((NQK, L, Dh), jnp.bfloat16),
        compiler_params=pltpu.CompilerParams(
            dimension_semantics=("parallel", "parallel")),
        interpret=interpret,
    )(y, normw3, cos2, sins2)


def _attn_call(qk, vaug, tri, interpret=False):
    f32 = jnp.float32
    bf16 = jnp.bfloat16
    return pl.pallas_call(
        _attn_kernel,
        grid=(Hq, L // BQ),
        in_specs=[
            pl.BlockSpec((1, BQ, Dh), lambda h, i: (h, i, 0)),
            pl.BlockSpec((1, L, Dh), lambda h, i: (Hq + h // REP, 0, 0)),
            pl.BlockSpec((1, L, VA), lambda h, i: (h // REP, 0, 0)),
            pl.BlockSpec((BQ, BK), lambda h, i: (0, 0)),
        ],
        out_specs=pl.BlockSpec((BQ, Dh), lambda h, i: (i, h)),
        out_shape=jax.ShapeDtypeStruct((L, Hq * Dh), bf16),
        scratch_shapes=[pltpu.VMEM((BQ, VA), f32),
                        pltpu.VMEM((BQ, BK), f32)],
        compiler_params=pltpu.CompilerParams(
            dimension_semantics=("parallel", "parallel")),
        interpret=interpret,
    )(qk, qk, vaug, tri)


def _oproj_call(o, wo_b, x2, nm2, Wgate, interpret=False):
    f32 = jnp.float32
    bf16 = jnp.bfloat16
    return pl.pallas_call(
        _oproj_gate_kernel,
        grid=(L // BT,),
        in_specs=[
            pl.BlockSpec((BT, Hq * Dh), lambda i: (i, 0)),
            pl.BlockSpec((Hq * Dh, Dm), lambda i: (0, 0)),
            pl.BlockSpec((BT, Dm), lambda i: (i, 0)),
            pl.BlockSpec((1, Dm), lambda i: (0, 0)),
            pl.BlockSpec((Dm, E), lambda i: (0, 0)),
        ],
        out_specs=[
            pl.BlockSpec((BT, Dm), lambda i: (i, 0)),
            pl.BlockSpec((BT, Dm), lambda i: (i, 0)),
            pl.BlockSpec((BT, E), lambda i: (i, 0)),
        ],
        out_shape=[
            jax.ShapeDtypeStruct((L, Dm), f32),
            jax.ShapeDtypeStruct((L, Dm), bf16),
            jax.ShapeDtypeStruct((L, E), f32),
        ],
        compiler_params=pltpu.CompilerParams(
            dimension_semantics=("parallel",)),
        interpret=interpret,
    )(o, wo_b, x2, nm2, Wgate)


def _moe_call(h2, wg_all, wu_all, wd_all, wfull, sel, xa, interpret=False):
    f32 = jnp.float32
    return pl.pallas_call(
        _moe_kernel,
        grid=(L // BTM,),
        in_specs=[
            pl.BlockSpec((BTM, Dm), lambda i: (i, 0)),
            pl.BlockSpec((Dm, E * F), lambda i: (0, 0)),
            pl.BlockSpec((Dm, E * F), lambda i: (0, 0)),
            pl.BlockSpec((E * F, Dm), lambda i: (0, 0)),
            pl.BlockSpec((BTM, E), lambda i: (i, 0)),
            pl.BlockSpec((E, E * F), lambda i: (0, 0)),
            pl.BlockSpec((BTM, Dm), lambda i: (i, 0)),
        ],
        out_specs=pl.BlockSpec((BTM, Dm), lambda i: (i, 0)),
        out_shape=jax.ShapeDtypeStruct((L, Dm), f32),
        compiler_params=pltpu.CompilerParams(
            dimension_semantics=("parallel",)),
        interpret=interpret,
    )(h2, wg_all, wu_all, wd_all, wfull, sel, xa)


@functools.partial(jax.jit, static_argnames=("interpret",))
def kernel(x, cos, sin, Wq, Wk, Wv, Wo, qn, kn, na, nm, Wgate, Wg, Wu, Wd,
           interpret=False):
    f32 = jnp.float32
    bf16 = jnp.bfloat16
    x2 = x[0]
    cos2 = cos[0]
    sign = jnp.concatenate([-jnp.ones((Dh // 2,), f32),
                            jnp.ones((Dh // 2,), f32)])
    sins2 = sin[0] * sign
    wqkv = jnp.concatenate([Wq, Wk, Wv], axis=1).astype(bf16)
    normw3 = jnp.concatenate(
        [jnp.tile(qn * (1.0 / math.sqrt(Dh)), Hq), jnp.tile(kn, Hkv)]
    ).reshape(NQK, 1, Dh)
    na2 = na.reshape(1, Dm)
    nm2 = nm.reshape(1, Dm)
    wo_b = Wo.astype(bf16)
    wg_all = jnp.transpose(Wg, (1, 0, 2)).reshape(Dm, E * F).astype(bf16)
    wu_all = jnp.transpose(Wu, (1, 0, 2)).reshape(Dm, E * F).astype(bf16)
    wd_all = Wd.reshape(E * F, Dm).astype(bf16)
    sel = jnp.repeat(jnp.eye(E, dtype=f32), F, axis=1).reshape(E, E * F)
    tri = jnp.tril(jnp.ones((BQ, BK), f32))

    hb = _rms_call(x2, na2, interpret=interpret)
    y = _qkvmm_call(hb, wqkv, interpret=interpret)
    qk = _rope_call(y, normw3, cos2, sins2, interpret=interpret)
    vv = y[NQK:]
    vaug = jnp.concatenate(
        [vv, jnp.ones((Hkv, L, 1), bf16),
         jnp.zeros((Hkv, L, VA - Dh - 1), bf16)], axis=2)
    o = _attn_call(qk, vaug, tri, interpret=interpret)
    xa, h2, wfull = _oproj_call(o, wo_b, x2, nm2, Wgate, interpret=interpret)
    out = _moe_call(h2, wg_all, wu_all, wd_all, wfull, sel, xa,
                    interpret=interpret)
    return out.reshape(B, L, Dm)


# speedup vs baseline: 1.1545x; 1.1136x over previous
import functools
import math

import jax
import jax.numpy as jnp
from jax.experimental import pallas as pl
from jax.experimental.pallas import tpu as pltpu

B, L, Dm, Hq, Hkv, Dh, E, K, F = 1, 2048, 2048, 32, 4, 128, 16, 8, 128
EPS = 1e-06
NH = Hq + 2 * Hkv
NQK = Hq + Hkv
REP = Hq // Hkv

BL = 1024
BQ = 512
BK = 1024
BT = 256
BTM = 512
VA = 2 * Dh


def _dot(a, b):
    return jax.lax.dot_general(a, b, (((1,), (0,)), ((), ())),
                               preferred_element_type=jnp.float32)


def _dot_nt(a, b):
    return jax.lax.dot_general(a, b, (((1,), (1,)), ((), ())),
                               preferred_element_type=jnp.float32)


def _rms_kernel(x_ref, na_ref, h_ref):
    x = x_ref[...]
    ms = jnp.mean(x * x, axis=-1, keepdims=True)
    h_ref[...] = (x * jax.lax.rsqrt(ms + EPS) * na_ref[...]
                  ).astype(jnp.bfloat16)


def _qkvmm_kernel(h_ref, w_ref, y_ref):
    y = _dot(h_ref[...], w_ref[...]).astype(jnp.bfloat16)
    for t in range(4):
        y_ref[t] = y[:, t * Dh:(t + 1) * Dh]


def _rope_kernel(y_ref, normw_ref, cos_ref, sins_ref, out_ref):
    y = y_ref[0].astype(jnp.float32)
    jm = jnp.full((Dh, Dh), 1.0 / Dh, jnp.bfloat16)
    msy = _dot((y * y).astype(jnp.bfloat16), jm)
    yn = y * jax.lax.rsqrt(msy + EPS) * normw_ref[0]
    rot = jnp.roll(yn, -(Dh // 2), axis=1)
    out_ref[0] = (cos_ref[...] * yn + sins_ref[...] * rot
                  ).astype(jnp.bfloat16)


def _attn_kernel(q_ref, k_ref, vaug_ref, mask_ref, o_ref, acc_scr, s_scr):
    i = pl.program_id(1)
    q = q_ref[0]
    acc_scr[...] = jnp.zeros((BQ, VA), jnp.float32)
    s_scr[...] = _dot_nt(q, k_ref[0, pl.ds(0, BK), :])
    nlast = (i * BQ + BQ + BK - 1) // BK - 1

    def body(j, _):
        p = jnp.exp(s_scr[...]).astype(jnp.bfloat16)
        vb = vaug_ref[0, pl.ds(j * BK, BK), :]
        kb1 = k_ref[0, pl.ds((j + 1) * BK, BK), :]
        acc_scr[...] += _dot(p, vb)
        s_scr[...] = _dot_nt(q, kb1)
        return 0

    jax.lax.fori_loop(0, nlast, body, 0)
    p = (jnp.exp(s_scr[...]) * mask_ref[0]).astype(jnp.bfloat16)
    acc = acc_scr[...] + _dot(p, vaug_ref[0, pl.ds(nlast * BK, BK), :])
    o_ref[...] = (acc[:, :Dh] / acc[:, Dh:Dh + 1]).astype(jnp.bfloat16)


def _oproj_gate_kernel(o_ref, wo_ref, x_ref, nm_ref, wg_ref,
                       xa_ref, h2_ref, wfull_ref):
    xa = x_ref[...] + _dot(o_ref[...], wo_ref[...])
    xa_ref[...] = xa
    ms = jnp.mean(xa * xa, axis=-1, keepdims=True)
    h2 = xa * jax.lax.rsqrt(ms + EPS) * nm_ref[...]
    h2_ref[...] = h2.astype(jnp.bfloat16)
    logits = _dot(h2, wg_ref[...])
    mx = jnp.max(logits, axis=-1, keepdims=True)
    ex = jnp.exp(logits - mx)
    probs = ex / jnp.sum(ex, axis=-1, keepdims=True)
    a = probs[:, None, :]
    b = probs[:, :, None]
    ij = jax.lax.broadcasted_iota(jnp.int32, (BT, E, E), 2)
    ie = jax.lax.broadcasted_iota(jnp.int32, (BT, E, E), 1)
    cmp = (a > b) | ((a == b) & (ij < ie))
    rank = jnp.sum(cmp.astype(jnp.float32), axis=2)
    w = jnp.where(rank < K, probs, 0.0)
    wfull_ref[...] = w / jnp.sum(w, axis=-1, keepdims=True)


def _moe_kernel(h2_ref, wg_ref, wu_ref, wd_ref, wfull_ref, sel_ref, xa_ref,
                out_ref):
    h2 = h2_ref[...]
    g = _dot(h2, wg_ref[...])
    u = _dot(h2, wu_ref[...])
    act = g * jax.lax.logistic(g) * u
    wexp = _dot(wfull_ref[...], sel_ref[...])
    down_in = (act * wexp).astype(jnp.bfloat16)
    out_ref[...] = xa_ref[...] + _dot(down_in, wd_ref[...])


def _rms_call(x2, na2, interpret=False):
    return pl.pallas_call(
        _rms_kernel,
        grid=(L // BL,),
        in_specs=[
            pl.BlockSpec((BL, Dm), lambda i: (i, 0)),
            pl.BlockSpec((1, Dm), lambda i: (0, 0)),
        ],
        out_specs=pl.BlockSpec((BL, Dm), lambda i: (i, 0)),
        out_shape=jax.ShapeDtypeStruct((L, Dm), jnp.bfloat16),
        compiler_params=pltpu.CompilerParams(
            dimension_semantics=("parallel",)),
        interpret=interpret,
    )(x2, na2)


def _qkvmm_call(hb, wqkv, interpret=False):
    return pl.pallas_call(
        _qkvmm_kernel,
        grid=(L // BL, NH // 4),
        in_specs=[
            pl.BlockSpec((BL, Dm), lambda i, j: (i, 0)),
            pl.BlockSpec((Dm, 4 * Dh), lambda i, j: (0, j)),
        ],
        out_specs=pl.BlockSpec((4, BL, Dh), lambda i, j: (j, i, 0)),
        out_shape=jax.ShapeDtypeStruct((NH, L, Dh), jnp.bfloat16),
        compiler_params=pltpu.CompilerParams(
            dimension_semantics=("parallel", "parallel")),
        interpret=interpret,
    )(hb, wqkv)


def _rope_call(y, normw3, cos2, sins2, interpret=False):
    return pl.pallas_call(
        _rope_kernel,
        grid=(L // BL, NQK),
        in_specs=[
            pl.BlockSpec((1, BL, Dh), lambda i, j: (j, i, 0)),
            pl.BlockSpec((1, 1, Dh), lambda i, j: (j, 0, 0)),
            pl.BlockSpec((BL, Dh), lambda i, j: (i, 0)),
            pl.BlockSpec((BL, Dh), lambda i, j: (i, 0)),
        ],
        out_specs=pl.BlockSpec((1, BL, Dh), lambda i, j: (j, i, 0)),
        out_shape=jax.ShapeDtypeStruct((NQK, L, Dh), jnp.bfloat16),
        compiler_params=pltpu.CompilerParams(
            dimension_semantics=("parallel", "parallel")),
        interpret=interpret,
    )(y, normw3, cos2, sins2)


def _attn_call(qk, vaug, masks, interpret=False):
    f32 = jnp.float32
    bf16 = jnp.bfloat16
    return pl.pallas_call(
        _attn_kernel,
        grid=(Hq, L // BQ),
        in_specs=[
            pl.BlockSpec((1, BQ, Dh), lambda h, i: (h, i, 0)),
            pl.BlockSpec((1, L, Dh), lambda h, i: (Hq + h // REP, 0, 0)),
            pl.BlockSpec((1, L, VA), lambda h, i: (h // REP, 0, 0)),
            pl.BlockSpec((1, BQ, BK), lambda h, i: (i % 2, 0, 0)),
        ],
        out_specs=pl.BlockSpec((BQ, Dh), lambda h, i: (i, h)),
        out_shape=jax.ShapeDtypeStruct((L, Hq * Dh), bf16),
        scratch_shapes=[pltpu.VMEM((BQ, VA), f32),
                        pltpu.VMEM((BQ, BK), f32)],
        compiler_params=pltpu.CompilerParams(
            dimension_semantics=("parallel", "parallel")),
        interpret=interpret,
    )(qk, qk, vaug, masks)


def _oproj_call(o, wo_b, x2, nm2, Wgate, interpret=False):
    f32 = jnp.float32
    bf16 = jnp.bfloat16
    return pl.pallas_call(
        _oproj_gate_kernel,
        grid=(L // BT,),
        in_specs=[
            pl.BlockSpec((BT, Hq * Dh), lambda i: (i, 0)),
            pl.BlockSpec((Hq * Dh, Dm), lambda i: (0, 0)),
            pl.BlockSpec((BT, Dm), lambda i: (i, 0)),
            pl.BlockSpec((1, Dm), lambda i: (0, 0)),
            pl.BlockSpec((Dm, E), lambda i: (0, 0)),
        ],
        out_specs=[
            pl.BlockSpec((BT, Dm), lambda i: (i, 0)),
            pl.BlockSpec((BT, Dm), lambda i: (i, 0)),
            pl.BlockSpec((BT, E), lambda i: (i, 0)),
        ],
        out_shape=[
            jax.ShapeDtypeStruct((L, Dm), f32),
            jax.ShapeDtypeStruct((L, Dm), bf16),
            jax.ShapeDtypeStruct((L, E), f32),
        ],
        compiler_params=pltpu.CompilerParams(
            dimension_semantics=("parallel",)),
        interpret=interpret,
    )(o, wo_b, x2, nm2, Wgate)


def _moe_call(h2, wg_all, wu_all, wd_all, wfull, sel, xa, interpret=False):
    f32 = jnp.float32
    return pl.pallas_call(
        _moe_kernel,
        grid=(L // BTM,),
        in_specs=[
            pl.BlockSpec((BTM, Dm), lambda i: (i, 0)),
            pl.BlockSpec((Dm, E * F), lambda i: (0, 0)),
            pl.BlockSpec((Dm, E * F), lambda i: (0, 0)),
            pl.BlockSpec((E * F, Dm), lambda i: (0, 0)),
            pl.BlockSpec((BTM, E), lambda i: (i, 0)),
            pl.BlockSpec((E, E * F), lambda i: (0, 0)),
            pl.BlockSpec((BTM, Dm), lambda i: (i, 0)),
        ],
        out_specs=pl.BlockSpec((BTM, Dm), lambda i: (i, 0)),
        out_shape=jax.ShapeDtypeStruct((L, Dm), f32),
        compiler_params=pltpu.CompilerParams(
            dimension_semantics=("parallel",)),
        interpret=interpret,
    )(h2, wg_all, wu_all, wd_all, wfull, sel, xa)


@functools.partial(jax.jit, static_argnames=("interpret",))
def kernel(x, cos, sin, Wq, Wk, Wv, Wo, qn, kn, na, nm, Wgate, Wg, Wu, Wd,
           interpret=False):
    f32 = jnp.float32
    bf16 = jnp.bfloat16
    x2 = x[0]
    cos2 = cos[0].astype(bf16)
    sign = jnp.concatenate([-jnp.ones((Dh // 2,), f32),
                            jnp.ones((Dh // 2,), f32)])
    sins2 = (sin[0] * sign).astype(bf16)
    wqkv = jnp.concatenate([Wq, Wk, Wv], axis=1).astype(bf16)
    normw3 = jnp.concatenate(
        [jnp.tile(qn * (1.0 / math.sqrt(Dh)), Hq), jnp.tile(kn, Hkv)]
    ).reshape(NQK, 1, Dh)
    na2 = na.reshape(1, Dm)
    nm2 = nm.reshape(1, Dm)
    wo_b = Wo.astype(bf16)
    wg_all = jnp.transpose(Wg, (1, 0, 2)).reshape(Dm, E * F).astype(bf16)
    wu_all = jnp.transpose(Wu, (1, 0, 2)).reshape(Dm, E * F).astype(bf16)
    wd_all = Wd.reshape(E * F, Dm).astype(bf16)
    sel = jnp.repeat(jnp.eye(E, dtype=f32), F, axis=1).reshape(E, E * F)
    half = jnp.tril(jnp.ones((BQ, BQ), f32))
    masks = jnp.stack([
        jnp.concatenate([half, jnp.zeros((BQ, BK - BQ), f32)], axis=1),
        jnp.concatenate([jnp.ones((BQ, BK - BQ), f32), half], axis=1),
    ])

    hb = _rms_call(x2, na2, interpret=interpret)
    y = _qkvmm_call(hb, wqkv, interpret=interpret)
    qk = _rope_call(y, normw3, cos2, sins2, interpret=interpret)
    vv = y[NQK:]
    vaug = jnp.concatenate(
        [vv, jnp.ones((Hkv, L, 1), bf16),
         jnp.zeros((Hkv, L, VA - Dh - 1), bf16)], axis=2)
    o = _attn_call(qk, vaug, masks, interpret=interpret)
    xa, h2, wfull = _oproj_call(o, wo_b, x2, nm2, Wgate, interpret=interpret)
    out = _moe_call(h2, wg_all, wu_all, wd_all, wfull, sel, xa,
                    interpret=interpret)
    return out.reshape(B, L, Dm)
